# baseline (speedup 1.0000x reference)
import functools

import jax
import jax.numpy as jnp
from jax import lax
from jax.experimental import pallas as pl
from jax.experimental.pallas import tpu as pltpu
from jax.experimental.pallas import tpu_sc as plsc

_ROWS = 1_000_000
_DIM = 64
_NC = 2
_NS = 16
_NW = _NC * _NS
_SLAB = 31_248
_TAIL_BASE = _SLAB * _NW

_mesh = plsc.VectorSubcoreMesh(core_axis_name="c", subcore_axis_name="s")


@functools.partial(
    pl.kernel,
    out_type=jax.ShapeDtypeStruct((_ROWS, _DIM), jnp.float32),
    mesh=_mesh,
)
def _sc_copy(in_hbm, out_hbm):
    wid = lax.axis_index("s") * _NC + lax.axis_index("c")
    base = pl.multiple_of(wid * _SLAB, 8)
    pltpu.sync_copy(
        in_hbm.at[pl.ds(base, _SLAB), :], out_hbm.at[pl.ds(base, _SLAB), :]
    )

    @pl.when(wid < 8)
    def _tail():
        tbase = pl.multiple_of(_TAIL_BASE + wid * 8, 8)
        pltpu.sync_copy(
            in_hbm.at[pl.ds(tbase, 8), :], out_hbm.at[pl.ds(tbase, 8), :]
        )


def kernel(path_emb):
    return _sc_copy(path_emb)


# speedup vs baseline: 15.4245x; 15.4245x over previous
import functools

import jax
import jax.numpy as jnp
from jax import lax
from jax.experimental import pallas as pl
from jax.experimental.pallas import tpu as pltpu
from jax.experimental.pallas import tpu_sc as plsc

_ROWS = 1_000_000
_DIM = 64
_NC = 2
_NS = 16
_NW = _NC * _NS
_CHUNK = 1_000
_NCHUNKS = _ROWS // _CHUNK
_MAX_PER_W = -(-_NCHUNKS // _NW)

_mesh = plsc.VectorSubcoreMesh(core_axis_name="c", subcore_axis_name="s")


@functools.partial(
    pl.kernel,
    out_type=jax.ShapeDtypeStruct((_ROWS, _DIM), jnp.float32),
    mesh=_mesh,
    scratch_types=[
        pltpu.VMEM((_CHUNK, _DIM), jnp.float32),
    ],
)
def _sc_copy(in_hbm, out_hbm, buf):
    wid = lax.axis_index("s") * _NC + lax.axis_index("c")

    def body(g, carry):
        c = wid + g * _NW

        @pl.when(c < _NCHUNKS)
        def _():
            base = pl.multiple_of(c * _CHUNK, 8)
            pltpu.sync_copy(in_hbm.at[pl.ds(base, _CHUNK), :], buf)
            pltpu.sync_copy(buf, out_hbm.at[pl.ds(base, _CHUNK), :])

        return carry

    lax.fori_loop(0, _MAX_PER_W, body, 0)


def kernel(path_emb):
    return _sc_copy(path_emb)
